# Initial kernel scaffold; baseline (speedup 1.0000x reference)
import jax
import jax.numpy as jnp
from jax.experimental import pallas as pl


def kernel(x, table, W, b):
    raise NotImplementedError("write your pallas kernel here")


# speedup vs baseline: 2.4641x; 2.4641x over previous
import functools

import jax
import jax.numpy as jnp
from jax import lax
from jax.experimental import pallas as pl
from jax.experimental.pallas import tpu as pltpu
from jax.experimental.pallas import tpu_sc as plsc

VOCAB = 1_000_000
DIM = 64
BATCH = 16384
HIST = 50

NW = 32
ROWS_PER_W = BATCH // NW
CB = 16
CHUNKS_PER_W = ROWS_PER_W // CB
NCHUNKS = BATCH // CB
IDX_PER_CHUNK = CB * HIST
GW = 80
NG = IDX_PER_CHUNK // GW
LANES = 16
KREG = DIM // LANES


def _sc_gather_pool(x3, table):
    mesh = plsc.VectorSubcoreMesh(core_axis_name="c", subcore_axis_name="s")
    nc = mesh.num_cores

    @functools.partial(
        pl.kernel,
        out_type=jax.ShapeDtypeStruct((NCHUNKS, CB, DIM), jnp.float32),
        mesh=mesh,
        scratch_types=[
            pltpu.VMEM((NG, GW), jnp.int32),
            pltpu.VMEM((IDX_PER_CHUNK, DIM), jnp.float32),
            pltpu.VMEM((CB, DIM), jnp.float32),
            pltpu.SemaphoreType.DMA,
        ],
        compiler_params=pltpu.CompilerParams(use_tc_tiling_on_sc=False),
    )
    def k(x_hbm, table_hbm, out_hbm, idx_v, rows_v, pooled_v, sem):
        wid = lax.axis_index("s") * nc + lax.axis_index("c")

        def chunk_body(g, carry):
            chunk = wid * CHUNKS_PER_W + g
            pltpu.sync_copy(x_hbm.at[chunk], idx_v)
            copies = [
                pltpu.async_copy(
                    table_hbm.at[idx_v.at[j]],
                    rows_v.at[pl.ds(j * GW, GW)],
                    sem,
                )
                for j in range(NG)
            ]
            for cpy in copies:
                cpy.wait()

            def row_body(bi, rcarry):
                base = bi * HIST
                accs = [rows_v[base, pl.ds(k16 * LANES, LANES)]
                        for k16 in range(KREG)]
                for l in range(1, HIST):
                    for k16 in range(KREG):
                        accs[k16] = accs[k16] + rows_v[
                            base + l, pl.ds(k16 * LANES, LANES)]
                for k16 in range(KREG):
                    pooled_v[bi, pl.ds(k16 * LANES, LANES)] = accs[k16]
                return rcarry

            lax.fori_loop(0, CB, row_body, 0)
            pltpu.sync_copy(pooled_v, out_hbm.at[chunk])
            return carry

        lax.fori_loop(0, CHUNKS_PER_W, chunk_body, 0)

    return k(x3, table)


def _tc_project(pooled, W, bvec):
    def mm(p_ref, w_ref, b_ref, o_ref):
        o_ref[...] = (
            jnp.dot(p_ref[...], w_ref[...],
                    preferred_element_type=jnp.float32) * (1.0 / HIST)
            + b_ref[...]
        )

    return pl.pallas_call(
        mm,
        out_shape=jax.ShapeDtypeStruct((BATCH, 2), jnp.float32),
    )(pooled, W, bvec)


def kernel(x, table, W, b):
    x3 = x.astype(jnp.int32).reshape(NCHUNKS, NG, GW)
    pooled = _sc_gather_pool(x3, table).reshape(BATCH, DIM)
    return _tc_project(pooled, W, b.reshape(1, 2))
